# baseline (speedup 1.0000x reference)
import functools

import jax
import jax.numpy as jnp
from jax import lax
from jax.experimental import pallas as pl
from jax.experimental.pallas import tpu as pltpu
from jax.experimental.pallas import tpu_sc as plsc

N = 10000
D_IN = 128
F = 4
N_CLASSES = 16

NP = 10112
E = 320000
CW = 128
CHUNKS = 80
EPW = CHUNKS * CW
EP = EPW * 32
ZR = NP // 16

_mesh = plsc.VectorSubcoreMesh(core_axis_name="c", subcore_axis_name="s")


def _make_edge_pass(with_gather):
    NBLK = 80
    BCH = CHUNKS // NBLK
    BR = BCH * CW
    NRING = 4

    @functools.partial(
        pl.kernel,
        mesh=_mesh,
        compiler_params=pltpu.CompilerParams(use_tc_tiling_on_sc=False),
        out_type=jax.ShapeDtypeStruct((2, NP, F), jnp.float32),
        scratch_types=[
            pltpu.VMEM((NBLK, BR), jnp.int32),
            pltpu.VMEM((CHUNKS, CW), jnp.int32),
            pltpu.VMEM((CHUNKS, CW, F), jnp.float32),
            pltpu.VMEM((ZR, F), jnp.float32),
            pltpu.VMEM_SHARED((NP, F), jnp.float32),
        ] + [pltpu.SemaphoreType.DMA] * (1 + NRING),
    )
    def _ep(row_hbm, col_hbm, g_hbm, z_hbm, out_hbm,
            rowv, colv, rowsv, zbuf, acc, *sems_all):
        ssem = sems_all[0]
        semg = sems_all[1:]
        cid = lax.axis_index("c")
        sid = lax.axis_index("s")
        wid = sid * 2 + cid

        def gfire2(k, b):
            pltpu.async_copy(g_hbm.at[rowv.at[k]], rowsv.at[k], semg[b])

        def gwait2(k, b):
            pltpu.make_async_copy(g_hbm.at[rowv.at[k]], rowsv.at[k],
                                  semg[b]).wait()

        pltpu.sync_copy(col_hbm.at[pl.ds(wid * CHUNKS, CHUNKS), :], colv)
        if with_gather:
            pltpu.sync_copy(row_hbm.at[pl.ds(wid * NBLK, NBLK), :], rowv)
            for j in range(NRING):
                gfire2(j, j)
        else:
            pltpu.sync_copy(g_hbm.at[pl.ds(0, CW), :], rowsv.at[0])
        pltpu.sync_copy(z_hbm.at[pl.ds(sid * ZR, ZR), :], zbuf)
        pltpu.sync_copy(zbuf, acc.at[pl.ds(sid * ZR, ZR), :])
        plsc.subcore_barrier()

        def scat(k):
            src = rowsv.at[k] if with_gather else rowsv.at[0]
            pltpu.async_copy(src, acc.at[colv.at[k]], ssem, add=True).wait()

        if with_gather:
            G = NBLK // NRING

            def gbody(gi, c):
                for b in range(NRING):
                    k = gi * NRING + b
                    gwait2(k, b)

                    @pl.when(gi < G - 1)
                    def _():
                        gfire2(k + NRING, b)
                    scat(k)
                return c

            lax.fori_loop(0, G, gbody, 0)
        else:
            def sbody(k, c):
                scat(k)
                return c

            lax.fori_loop(0, CHUNKS, sbody, 0)
        plsc.subcore_barrier()
        pltpu.sync_copy(acc.at[pl.ds(sid * ZR, ZR), :], zbuf)
        pltpu.sync_copy(zbuf, out_hbm.at[cid, pl.ds(sid * ZR, ZR), :])

    return _ep


_edge_pass = _make_edge_pass(True)
_deg_pass = _make_edge_pass(False)


def _t1_body(dp, x, w1, dis_o, g1_o):
    deg = dp[0, :N, 0:1] + dp[1, :N, 0:1] + 1.0
    dis = lax.rsqrt(deg)
    dis_o[...] = dis
    z = jnp.dot(x[...], w1[...], preferred_element_type=jnp.float32)
    g1_o[...] = z * dis


_t1 = pl.pallas_call(
    _t1_body,
    out_shape=(
        jax.ShapeDtypeStruct((N, 1), jnp.float32),
        jax.ShapeDtypeStruct((N, F), jnp.float32),
    ),
)


def _mid_stage(fo):
    def body(sp, g, dis, b, w, gout):
        s = sp[0, :N, :] + sp[1, :N, :] + g[...]
        h = jnp.tanh(dis[...] * s + b[...])
        z = jnp.dot(h, w[...], preferred_element_type=jnp.float32)
        gz = dis[...] * z
        if fo < F:
            gz = jnp.concatenate([gz, jnp.zeros((N, F - fo), jnp.float32)], axis=1)
        gout[...] = gz

    return pl.pallas_call(
        body,
        out_shape=jax.ShapeDtypeStruct((N, F), jnp.float32),
    )


_t2 = _mid_stage(F)
_t3 = _mid_stage(2)


def _t4_body(sp, g3, dis, b3, wc, bc, out_o, h3_o):
    s = sp[0, :N, 0:2] + sp[1, :N, 0:2] + g3[:, 0:2]
    h3 = jnp.tanh(dis[...] * s + b3[...])
    h3_o[...] = h3
    out_o[...] = jnp.dot(h3, wc[...], preferred_element_type=jnp.float32) + bc[...]


_t4 = pl.pallas_call(
    _t4_body,
    out_shape=(
        jax.ShapeDtypeStruct((N, N_CLASSES), jnp.float32),
        jax.ShapeDtypeStruct((N, 2), jnp.float32),
    ),
)


def kernel(x, edge_index, W1, b1, W2, b2, W3, b3, Wc, bc):
    row = edge_index[0]
    col = edge_index[1]
    pad = EP - E
    row2d = jnp.concatenate([row, jnp.zeros((pad,), jnp.int32)]).reshape(-1, CW)
    col2d = jnp.concatenate([col, jnp.full((pad,), N, jnp.int32)]).reshape(-1, CW)
    zeros_np = jnp.zeros((NP, F), jnp.float32)
    ones_tab = jnp.ones((CW, F), jnp.float32)

    deg_p = _deg_pass(row2d, col2d, ones_tab, zeros_np)
    dis, g1 = _t1(deg_p, x, W1)
    s1 = _edge_pass(row2d, col2d, g1, zeros_np)
    g2 = _t2(s1, g1, dis, b1.reshape(1, F), W2)
    s2 = _edge_pass(row2d, col2d, g2, zeros_np)
    g3 = _t3(s2, g2, dis, b2.reshape(1, F), W3)
    s3 = _edge_pass(row2d, col2d, g3, zeros_np)
    out, h3 = _t4(s3, g3, dis, b3.reshape(1, 2), Wc, bc.reshape(1, N_CLASSES))
    return (out, h3)


# speedup vs baseline: 1.0047x; 1.0047x over previous
import functools

import jax
import jax.numpy as jnp
from jax import lax
from jax.experimental import pallas as pl
from jax.experimental.pallas import tpu as pltpu
from jax.experimental.pallas import tpu_sc as plsc

N = 10000
D_IN = 128
F = 4
N_CLASSES = 16

NP = 10112
E = 320000
CW = 128
CHUNKS = 80
EPW = CHUNKS * CW
EP = EPW * 32
ZR = NP // 16

_mesh = plsc.VectorSubcoreMesh(core_axis_name="c", subcore_axis_name="s")


def _make_edge_pass(with_gather):
    NBLK = 80
    BCH = CHUNKS // NBLK
    BR = BCH * CW
    NRING = 4

    @functools.partial(
        pl.kernel,
        mesh=_mesh,
        compiler_params=pltpu.CompilerParams(use_tc_tiling_on_sc=False),
        out_type=jax.ShapeDtypeStruct((2, NP, F), jnp.float32),
        scratch_types=[
            pltpu.VMEM((NBLK, BR), jnp.int32),
            pltpu.VMEM((CHUNKS, CW), jnp.int32),
            pltpu.VMEM((CHUNKS, CW, F), jnp.float32),
            pltpu.VMEM((ZR, F), jnp.float32),
            pltpu.VMEM_SHARED((NP, F), jnp.float32),
        ] + [pltpu.SemaphoreType.DMA] * (1 + NRING),
    )
    def _ep(row_hbm, col_hbm, g_hbm, z_hbm, out_hbm,
            rowv, colv, rowsv, zbuf, acc, *sems_all):
        ssem = sems_all[0]
        semg = sems_all[1:]
        cid = lax.axis_index("c")
        sid = lax.axis_index("s")
        wid = sid * 2 + cid

        def gfire2(k, b):
            pltpu.async_copy(g_hbm.at[rowv.at[k]], rowsv.at[k], semg[b])

        def gwait2(k, b):
            pltpu.make_async_copy(g_hbm.at[rowv.at[k]], rowsv.at[k],
                                  semg[b]).wait()

        pltpu.sync_copy(col_hbm.at[pl.ds(wid * CHUNKS, CHUNKS), :], colv)
        if with_gather:
            pltpu.sync_copy(row_hbm.at[pl.ds(wid * NBLK, NBLK), :], rowv)
            for j in range(NRING):
                gfire2(j, j)
        else:
            pltpu.sync_copy(g_hbm.at[pl.ds(0, CW), :], rowsv.at[0])
        pltpu.sync_copy(z_hbm.at[pl.ds(sid * ZR, ZR), :], zbuf)
        pltpu.sync_copy(zbuf, acc.at[pl.ds(sid * ZR, ZR), :])
        plsc.subcore_barrier()

        def scat(k):
            src = rowsv.at[k] if with_gather else rowsv.at[0]
            pltpu.async_copy(src, acc.at[colv.at[k]], ssem, add=True).wait()

        if with_gather:
            G = NBLK // NRING

            def gbody(gi, c):
                for b in range(NRING):
                    k = gi * NRING + b
                    gwait2(k, b)
                    scat(k)

                    @pl.when(gi < G - 1)
                    def _():
                        gfire2(k + NRING, b)
                return c

            lax.fori_loop(0, G, gbody, 0)
        else:
            def sbody(k, c):
                scat(k)
                return c

            lax.fori_loop(0, CHUNKS, sbody, 0)
        plsc.subcore_barrier()
        pltpu.sync_copy(acc.at[pl.ds(sid * ZR, ZR), :], zbuf)
        pltpu.sync_copy(zbuf, out_hbm.at[cid, pl.ds(sid * ZR, ZR), :])

    return _ep


_edge_pass = _make_edge_pass(True)
_deg_pass = _make_edge_pass(False)


def _t1_body(dp, x, w1, dis_o, g1_o):
    deg = dp[0, :N, 0:1] + dp[1, :N, 0:1] + 1.0
    dis = lax.rsqrt(deg)
    dis_o[...] = dis
    z = jnp.dot(x[...], w1[...], preferred_element_type=jnp.float32)
    g1_o[...] = z * dis


_t1 = pl.pallas_call(
    _t1_body,
    out_shape=(
        jax.ShapeDtypeStruct((N, 1), jnp.float32),
        jax.ShapeDtypeStruct((N, F), jnp.float32),
    ),
)


def _mid_stage(fo):
    def body(sp, g, dis, b, w, gout):
        s = sp[0, :N, :] + sp[1, :N, :] + g[...]
        h = jnp.tanh(dis[...] * s + b[...])
        z = jnp.dot(h, w[...], preferred_element_type=jnp.float32)
        gz = dis[...] * z
        if fo < F:
            gz = jnp.concatenate([gz, jnp.zeros((N, F - fo), jnp.float32)], axis=1)
        gout[...] = gz

    return pl.pallas_call(
        body,
        out_shape=jax.ShapeDtypeStruct((N, F), jnp.float32),
    )


_t2 = _mid_stage(F)
_t3 = _mid_stage(2)


def _t4_body(sp, g3, dis, b3, wc, bc, out_o, h3_o):
    s = sp[0, :N, 0:2] + sp[1, :N, 0:2] + g3[:, 0:2]
    h3 = jnp.tanh(dis[...] * s + b3[...])
    h3_o[...] = h3
    out_o[...] = jnp.dot(h3, wc[...], preferred_element_type=jnp.float32) + bc[...]


_t4 = pl.pallas_call(
    _t4_body,
    out_shape=(
        jax.ShapeDtypeStruct((N, N_CLASSES), jnp.float32),
        jax.ShapeDtypeStruct((N, 2), jnp.float32),
    ),
)


def kernel(x, edge_index, W1, b1, W2, b2, W3, b3, Wc, bc):
    row = edge_index[0]
    col = edge_index[1]
    pad = EP - E
    row2d = jnp.concatenate([row, jnp.zeros((pad,), jnp.int32)]).reshape(-1, CW)
    col2d = jnp.concatenate([col, jnp.full((pad,), N, jnp.int32)]).reshape(-1, CW)
    zeros_np = jnp.zeros((NP, F), jnp.float32)
    ones_tab = jnp.ones((CW, F), jnp.float32)

    deg_p = _deg_pass(row2d, col2d, ones_tab, zeros_np)
    dis, g1 = _t1(deg_p, x, W1)
    s1 = _edge_pass(row2d, col2d, g1, zeros_np)
    g2 = _t2(s1, g1, dis, b1.reshape(1, F), W2)
    s2 = _edge_pass(row2d, col2d, g2, zeros_np)
    g3 = _t3(s2, g2, dis, b2.reshape(1, F), W3)
    s3 = _edge_pass(row2d, col2d, g3, zeros_np)
    out, h3 = _t4(s3, g3, dis, b3.reshape(1, 2), Wc, bc.reshape(1, N_CLASSES))
    return (out, h3)


# speedup vs baseline: 1.0396x; 1.0347x over previous
import functools

import jax
import jax.numpy as jnp
from jax import lax
from jax.experimental import pallas as pl
from jax.experimental.pallas import tpu as pltpu
from jax.experimental.pallas import tpu_sc as plsc

N = 10000
D_IN = 128
F = 4
N_CLASSES = 16

NP = 10112
E = 320000
CW = 128
CHUNKS = 80
EPW = CHUNKS * CW
EP = EPW * 32
ZR = NP // 16
C_FAST = 104
C_SLOW = 56

_mesh = plsc.VectorSubcoreMesh(core_axis_name="c", subcore_axis_name="s")


def _make_edge_pass(with_gather):
    NRING = 4
    CA, CB = C_FAST, C_SLOW
    CMAX = max(CA, CB)
    CB_BASE = 16 * CA

    @functools.partial(
        pl.kernel,
        mesh=_mesh,
        compiler_params=pltpu.CompilerParams(use_tc_tiling_on_sc=False),
        out_type=jax.ShapeDtypeStruct((2, NP, F), jnp.float32),
        scratch_types=[
            pltpu.VMEM((CMAX, CW), jnp.int32),
            pltpu.VMEM((CMAX, CW), jnp.int32),
            pltpu.VMEM((NRING, CW, F), jnp.float32),
            pltpu.VMEM((ZR, F), jnp.float32),
            pltpu.VMEM_SHARED((NP, F), jnp.float32),
        ] + [pltpu.SemaphoreType.DMA] * (1 + NRING),
    )
    def _ep(row_hbm, col_hbm, g_hbm, z_hbm, out_hbm,
            rowv, colv, rowsv, zbuf, acc, *sems_all):
        ssem = sems_all[0]
        semg = sems_all[1:]
        cid = lax.axis_index("c")
        sid = lax.axis_index("s")

        def gfire2(k, b):
            pltpu.async_copy(g_hbm.at[rowv.at[k]], rowsv.at[b], semg[b])

        def gwait2(k, b):
            pltpu.make_async_copy(g_hbm.at[rowv.at[k]], rowsv.at[b],
                                  semg[b]).wait()

        def stage_and_prime(C, cbase):
            pltpu.sync_copy(col_hbm.at[pl.ds(cbase, C), :],
                            colv.at[pl.ds(0, C), :])
            if with_gather:
                pltpu.sync_copy(row_hbm.at[pl.ds(cbase, C), :],
                                rowv.at[pl.ds(0, C), :])
                for j in range(NRING):
                    gfire2(j, j)

        @pl.when(cid == 0)
        def _():
            stage_and_prime(CA, sid * CA)

        @pl.when(cid == 1)
        def _():
            stage_and_prime(CB, CB_BASE + sid * CB)

        if not with_gather:
            pltpu.sync_copy(g_hbm.at[pl.ds(0, CW), :], rowsv.at[0])
        pltpu.sync_copy(z_hbm.at[pl.ds(sid * ZR, ZR), :], zbuf)
        pltpu.sync_copy(zbuf, acc.at[pl.ds(sid * ZR, ZR), :])
        plsc.subcore_barrier()

        def scat(k, b):
            src = rowsv.at[b] if with_gather else rowsv.at[0]
            pltpu.async_copy(src, acc.at[colv.at[k]], ssem, add=True).wait()

        def mainloop(C):
            if with_gather:
                G = C // NRING

                def gbody(gi, c):
                    for b in range(NRING):
                        k = gi * NRING + b
                        gwait2(k, b)
                        scat(k, b)

                        @pl.when(gi < G - 1)
                        def _():
                            gfire2(k + NRING, b)
                    return c

                lax.fori_loop(0, G, gbody, 0)
            else:
                def sbody(k, c):
                    scat(k, 0)
                    return c

                lax.fori_loop(0, C, sbody, 0)

        @pl.when(cid == 0)
        def _():
            mainloop(CA)

        @pl.when(cid == 1)
        def _():
            mainloop(CB)

        plsc.subcore_barrier()
        pltpu.sync_copy(acc.at[pl.ds(sid * ZR, ZR), :], zbuf)
        pltpu.sync_copy(zbuf, out_hbm.at[cid, pl.ds(sid * ZR, ZR), :])

    return _ep


_edge_pass = _make_edge_pass(True)
_deg_pass = _make_edge_pass(False)


def _t1_body(dp, x, w1, dis_o, g1_o):
    deg = dp[0, :N, 0:1] + dp[1, :N, 0:1] + 1.0
    dis = lax.rsqrt(deg)
    dis_o[...] = dis
    z = jnp.dot(x[...], w1[...], preferred_element_type=jnp.float32)
    g1_o[...] = z * dis


_t1 = pl.pallas_call(
    _t1_body,
    out_shape=(
        jax.ShapeDtypeStruct((N, 1), jnp.float32),
        jax.ShapeDtypeStruct((N, F), jnp.float32),
    ),
)


def _mid_stage(fo):
    def body(sp, g, dis, b, w, gout):
        s = sp[0, :N, :] + sp[1, :N, :] + g[...]
        h = jnp.tanh(dis[...] * s + b[...])
        z = jnp.dot(h, w[...], preferred_element_type=jnp.float32)
        gz = dis[...] * z
        if fo < F:
            gz = jnp.concatenate([gz, jnp.zeros((N, F - fo), jnp.float32)], axis=1)
        gout[...] = gz

    return pl.pallas_call(
        body,
        out_shape=jax.ShapeDtypeStruct((N, F), jnp.float32),
    )


_t2 = _mid_stage(F)
_t3 = _mid_stage(2)


def _t4_body(sp, g3, dis, b3, wc, bc, out_o, h3_o):
    s = sp[0, :N, 0:2] + sp[1, :N, 0:2] + g3[:, 0:2]
    h3 = jnp.tanh(dis[...] * s + b3[...])
    h3_o[...] = h3
    out_o[...] = jnp.dot(h3, wc[...], preferred_element_type=jnp.float32) + bc[...]


_t4 = pl.pallas_call(
    _t4_body,
    out_shape=(
        jax.ShapeDtypeStruct((N, N_CLASSES), jnp.float32),
        jax.ShapeDtypeStruct((N, 2), jnp.float32),
    ),
)


def kernel(x, edge_index, W1, b1, W2, b2, W3, b3, Wc, bc):
    row = edge_index[0]
    col = edge_index[1]
    pad = EP - E
    row2d = jnp.concatenate([row, jnp.zeros((pad,), jnp.int32)]).reshape(-1, CW)
    col2d = jnp.concatenate([col, jnp.full((pad,), N, jnp.int32)]).reshape(-1, CW)
    zeros_np = jnp.zeros((NP, F), jnp.float32)
    ones_tab = jnp.ones((CW, F), jnp.float32)

    deg_p = _deg_pass(row2d, col2d, ones_tab, zeros_np)
    dis, g1 = _t1(deg_p, x, W1)
    s1 = _edge_pass(row2d, col2d, g1, zeros_np)
    g2 = _t2(s1, g1, dis, b1.reshape(1, F), W2)
    s2 = _edge_pass(row2d, col2d, g2, zeros_np)
    g3 = _t3(s2, g2, dis, b2.reshape(1, F), W3)
    s3 = _edge_pass(row2d, col2d, g3, zeros_np)
    out, h3 = _t4(s3, g3, dis, b3.reshape(1, 2), Wc, bc.reshape(1, N_CLASSES))
    return (out, h3)
